# baseline (speedup 1.0000x reference)
import functools

import jax
import jax.numpy as jnp
from jax import lax
from jax.experimental import pallas as pl
from jax.experimental.pallas import tpu as pltpu
from jax.experimental.pallas import tpu_sc as plsc

N = 2048
D = 768
H = 3072
E = 8
K = 2
B = 128
P = N * K + E * B
NB = P // B
NW = 32
L = 16


def _gelu_tanh(x):
    return 0.5 * x * (1.0 + jnp.tanh(jnp.sqrt(2.0 / jnp.pi) * (x + 0.044715 * x ** 3)))


def _gate_body(x_ref, wg_ref, bg_ref,
               pos0_ref, pos1_ref, w0_ref, w1_ref, ob_ref, nb_ref):
    logits = lax.dot_general(
        wg_ref[...], x_ref[...], (((0,), (1,)), ((), ())),
        preferred_element_type=jnp.float32) + bg_ref[...]
    m = jnp.max(logits, axis=0, keepdims=True)
    ex = jnp.exp(logits - m)
    p = ex / jnp.sum(ex, axis=0, keepdims=True)

    iota8 = lax.broadcasted_iota(jnp.int32, (E, N), 0)
    m1 = jnp.max(p, axis=0, keepdims=True)
    a1 = jnp.min(jnp.where(p == m1, iota8, E), axis=0, keepdims=True)
    oh1 = iota8 == a1
    pm = jnp.where(oh1, -1.0, p)
    m2 = jnp.max(pm, axis=0, keepdims=True)
    a2 = jnp.min(jnp.where(pm == m2, iota8, E), axis=0, keepdims=True)
    oh2 = iota8 == a2

    w0_ref[...] = jnp.sum(jnp.where(oh1, p, 0.0), axis=0, keepdims=True)
    w1_ref[...] = jnp.sum(jnp.where(oh2, p, 0.0), axis=0, keepdims=True)

    oh = oh1.astype(jnp.int32) + oh2.astype(jnp.int32)
    c = oh
    k = 1
    while k < N:
        c = c + jnp.concatenate(
            [jnp.zeros((E, k), jnp.int32), c[:, :N - k]], axis=1)
        k *= 2
    cexc = c - oh
    counts = c[:, N - 1:N]

    pc = ((counts + (B - 1)) // B) * B
    ends = pc
    k = 1
    while k < E:
        ends = ends + jnp.concatenate(
            [jnp.zeros((k, 1), jnp.int32), ends[:E - k, :]], axis=0)
        k *= 2
    opad = ends - pc

    r0 = jnp.sum(jnp.where(oh1, cexc, 0), axis=0, keepdims=True)
    r1 = jnp.sum(jnp.where(oh2, cexc, 0), axis=0, keepdims=True)
    off0 = jnp.sum(jnp.where(oh1, opad, 0), axis=0, keepdims=True)
    off1 = jnp.sum(jnp.where(oh2, opad, 0), axis=0, keepdims=True)
    pos0_ref[...] = off0 + r0
    pos1_ref[...] = off1 + r1

    ob_ref[...] = opad // B
    nb_ref[...] = pc // B


def _moe_body(ob_ref, nb_ref, p0_ref, p1_ref, w0_ref, w1_ref, x_ref,
              w1e_ref, b1_ref, w2e_ref, b2_ref, ys_ref, ybuf, sems):
    e = pl.program_id(0)
    blk0 = ob_ref[e]
    nblk = nb_ref[e]

    def compute_block(j, slot):
        sid = (blk0 + j) * B + lax.broadcasted_iota(jnp.int32, (B, 1), 0)
        m0 = p0_ref[...] == sid
        m1 = p1_ref[...] == sid
        gm = (m0 | m1).astype(jnp.float32)
        xb = lax.dot_general(
            gm, x_ref[...], (((1,), (0,)), ((), ())),
            preferred_element_type=jnp.float32)
        sw = jnp.sum(jnp.where(m0, w0_ref[...], 0.0) +
                     jnp.where(m1, w1_ref[...], 0.0), axis=1, keepdims=True)
        h = lax.dot_general(
            xb, w1e_ref[0], (((1,), (0,)), ((), ())),
            preferred_element_type=jnp.float32) + b1_ref[0]
        h = _gelu_tanh(h)
        y = lax.dot_general(
            h, w2e_ref[0], (((1,), (0,)), ((), ())),
            preferred_element_type=jnp.float32) + b2_ref[0]
        ybuf[slot] = y * sw
        pltpu.make_async_copy(
            ybuf.at[slot], ys_ref.at[pl.ds((blk0 + j) * B, B)],
            sems.at[slot]).start()

    def pair_body(t, carry):
        for s in range(2):
            j = 2 * t + s

            @pl.when(j < nblk)
            def _():
                @pl.when(j >= 2)
                def _w():
                    pltpu.make_async_copy(
                        ybuf.at[s], ys_ref.at[pl.ds(0, B)], sems.at[s]).wait()
                compute_block(j, s)
        return carry

    lax.fori_loop(0, (nblk + 1) // 2, pair_body, 0)

    @pl.when(nblk >= 1)
    def _d0():
        pltpu.make_async_copy(
            ybuf.at[0], ys_ref.at[pl.ds(0, B)], sems.at[0]).wait()

    @pl.when(nblk >= 2)
    def _d1():
        pltpu.make_async_copy(
            ybuf.at[1], ys_ref.at[pl.ds(0, B)], sems.at[1]).wait()


_T_PER_W = N // NW


def _combine_body(ys_hbm, p0_hbm, p1_hbm, out_hbm,
                  i0_v, i1_v, r0_v, r1_v, s0, s1):
    wid = lax.axis_index("s") * 2 + lax.axis_index("c")
    base = wid * _T_PER_W
    pltpu.sync_copy(p0_hbm.at[0, pl.ds(base, _T_PER_W)], i0_v)
    pltpu.sync_copy(p1_hbm.at[0, pl.ds(base, _T_PER_W)], i1_v)
    cp0 = pltpu.async_copy(ys_hbm.at[i0_v], r0_v, s0)
    cp1 = pltpu.async_copy(ys_hbm.at[i1_v], r1_v, s1)
    cp0.wait()
    cp1.wait()

    def tbody(t, carry):
        for j in range(D // L):
            sl = pl.ds(j * L, L)
            r0_v[t, sl] = r0_v[t, sl] + r1_v[t, sl]
        return carry
    lax.fori_loop(0, _T_PER_W, tbody, 0)

    pltpu.sync_copy(r0_v, out_hbm.at[pl.ds(base, _T_PER_W)])


def kernel(x, Wg, bg, W1, b1, W2, b2):
    f32 = jnp.float32
    i32 = jnp.int32

    pos0, pos1, w0, w1, ob, nb = pl.pallas_call(
        _gate_body,
        out_shape=[
            jax.ShapeDtypeStruct((1, N), i32),
            jax.ShapeDtypeStruct((1, N), i32),
            jax.ShapeDtypeStruct((1, N), f32),
            jax.ShapeDtypeStruct((1, N), f32),
            jax.ShapeDtypeStruct((E, 1), i32),
            jax.ShapeDtypeStruct((E, 1), i32),
        ],
    )(x, Wg, bg.reshape(E, 1))

    grid_spec = pltpu.PrefetchScalarGridSpec(
        num_scalar_prefetch=2,
        grid=(E,),
        in_specs=[
            pl.BlockSpec((1, N), lambda e, ob_s, nb_s: (0, 0)),
            pl.BlockSpec((1, N), lambda e, ob_s, nb_s: (0, 0)),
            pl.BlockSpec((1, N), lambda e, ob_s, nb_s: (0, 0)),
            pl.BlockSpec((1, N), lambda e, ob_s, nb_s: (0, 0)),
            pl.BlockSpec((N, D), lambda e, ob_s, nb_s: (0, 0)),
            pl.BlockSpec((1, D, H), lambda e, ob_s, nb_s: (e, 0, 0)),
            pl.BlockSpec((1, 1, H), lambda e, ob_s, nb_s: (e, 0, 0)),
            pl.BlockSpec((1, H, D), lambda e, ob_s, nb_s: (e, 0, 0)),
            pl.BlockSpec((1, 1, D), lambda e, ob_s, nb_s: (e, 0, 0)),
        ],
        out_specs=pl.BlockSpec(memory_space=pltpu.MemorySpace.HBM),
        scratch_shapes=[
            pltpu.VMEM((2, B, D), f32),
            pltpu.SemaphoreType.DMA((2,)),
        ],
    )
    ys = pl.pallas_call(
        _moe_body,
        grid_spec=grid_spec,
        out_shape=jax.ShapeDtypeStruct((P, D), f32),
    )(ob.reshape(E), nb.reshape(E), pos0, pos1, w0, w1, x,
      W1, b1.reshape(E, 1, H), W2, b2.reshape(E, 1, D))

    mesh = plsc.VectorSubcoreMesh(core_axis_name="c", subcore_axis_name="s")
    combine_k = pl.kernel(
        _combine_body,
        out_type=jax.ShapeDtypeStruct((N, D), f32),
        mesh=mesh,
        scratch_types=[
            pltpu.VMEM((_T_PER_W,), i32),
            pltpu.VMEM((_T_PER_W,), i32),
            pltpu.VMEM((_T_PER_W, D), f32),
            pltpu.VMEM((_T_PER_W, D), f32),
            pltpu.SemaphoreType.DMA,
            pltpu.SemaphoreType.DMA,
        ],
        compiler_params=pltpu.CompilerParams(needs_layout_passes=False),
    )
    return combine_k(ys, pos0, pos1)


# speedup vs baseline: 1.0237x; 1.0237x over previous
import functools

import jax
import jax.numpy as jnp
from jax import lax
from jax.experimental import pallas as pl
from jax.experimental.pallas import tpu as pltpu
from jax.experimental.pallas import tpu_sc as plsc

N = 2048
D = 768
H = 3072
E = 8
K = 2
B = 256
P = N * K + E * B
NB = P // B
NW = 32
L = 16


def _gelu_tanh(x):
    return 0.5 * x * (1.0 + jnp.tanh(jnp.sqrt(2.0 / jnp.pi) * (x + 0.044715 * x ** 3)))


def _gate_body(x_ref, wg_ref, bg_ref,
               pos0_ref, pos1_ref, w0_ref, w1_ref, ob_ref, nb_ref):
    logits = lax.dot_general(
        wg_ref[...], x_ref[...], (((0,), (1,)), ((), ())),
        preferred_element_type=jnp.float32) + bg_ref[...]
    m = jnp.max(logits, axis=0, keepdims=True)
    ex = jnp.exp(logits - m)
    p = ex / jnp.sum(ex, axis=0, keepdims=True)

    iota8 = lax.broadcasted_iota(jnp.int32, (E, N), 0)
    m1 = jnp.max(p, axis=0, keepdims=True)
    a1 = jnp.min(jnp.where(p == m1, iota8, E), axis=0, keepdims=True)
    oh1 = iota8 == a1
    pm = jnp.where(oh1, -1.0, p)
    m2 = jnp.max(pm, axis=0, keepdims=True)
    a2 = jnp.min(jnp.where(pm == m2, iota8, E), axis=0, keepdims=True)
    oh2 = iota8 == a2

    w0_ref[...] = jnp.sum(jnp.where(oh1, p, 0.0), axis=0, keepdims=True)
    w1_ref[...] = jnp.sum(jnp.where(oh2, p, 0.0), axis=0, keepdims=True)

    oh = oh1.astype(jnp.int32) + oh2.astype(jnp.int32)
    c = oh
    k = 1
    while k < N:
        c = c + jnp.concatenate(
            [jnp.zeros((E, k), jnp.int32), c[:, :N - k]], axis=1)
        k *= 2
    cexc = c - oh
    counts = c[:, N - 1:N]

    pc = ((counts + (B - 1)) // B) * B
    ends = pc
    k = 1
    while k < E:
        ends = ends + jnp.concatenate(
            [jnp.zeros((k, 1), jnp.int32), ends[:E - k, :]], axis=0)
        k *= 2
    opad = ends - pc

    r0 = jnp.sum(jnp.where(oh1, cexc, 0), axis=0, keepdims=True)
    r1 = jnp.sum(jnp.where(oh2, cexc, 0), axis=0, keepdims=True)
    off0 = jnp.sum(jnp.where(oh1, opad, 0), axis=0, keepdims=True)
    off1 = jnp.sum(jnp.where(oh2, opad, 0), axis=0, keepdims=True)
    pos0_ref[...] = off0 + r0
    pos1_ref[...] = off1 + r1

    ob_ref[...] = opad // B
    nb_ref[...] = pc // B


def _moe_body(ob_ref, nb_ref, p0_ref, p1_ref, w0_ref, w1_ref, x_ref,
              w1e_ref, b1_ref, w2e_ref, b2_ref, ys_ref, ybuf, sems):
    e = pl.program_id(0)
    blk0 = ob_ref[e]
    nblk = nb_ref[e]

    def compute_block(j, slot):
        sid = (blk0 + j) * B + lax.broadcasted_iota(jnp.int32, (B, 1), 0)
        m0 = p0_ref[...] == sid
        m1 = p1_ref[...] == sid
        gm = (m0 | m1).astype(jnp.float32)
        xb = lax.dot_general(
            gm, x_ref[...], (((1,), (0,)), ((), ())),
            preferred_element_type=jnp.float32)
        sw = jnp.sum(jnp.where(m0, w0_ref[...], 0.0) +
                     jnp.where(m1, w1_ref[...], 0.0), axis=1, keepdims=True)
        h = lax.dot_general(
            xb, w1e_ref[0], (((1,), (0,)), ((), ())),
            preferred_element_type=jnp.float32) + b1_ref[0]
        h = _gelu_tanh(h)
        y = lax.dot_general(
            h, w2e_ref[0], (((1,), (0,)), ((), ())),
            preferred_element_type=jnp.float32) + b2_ref[0]
        ybuf[slot] = y * sw
        pltpu.make_async_copy(
            ybuf.at[slot], ys_ref.at[pl.ds((blk0 + j) * B, B)],
            sems.at[slot]).start()

    def pair_body(t, carry):
        for s in range(2):
            j = 2 * t + s

            @pl.when(j < nblk)
            def _():
                @pl.when(j >= 2)
                def _w():
                    pltpu.make_async_copy(
                        ybuf.at[s], ys_ref.at[pl.ds(0, B)], sems.at[s]).wait()
                compute_block(j, s)
        return carry

    lax.fori_loop(0, (nblk + 1) // 2, pair_body, 0)

    @pl.when(nblk >= 1)
    def _d0():
        pltpu.make_async_copy(
            ybuf.at[0], ys_ref.at[pl.ds(0, B)], sems.at[0]).wait()

    @pl.when(nblk >= 2)
    def _d1():
        pltpu.make_async_copy(
            ybuf.at[1], ys_ref.at[pl.ds(0, B)], sems.at[1]).wait()


_T_PER_W = N // NW


def _combine_body(ys_hbm, p0_hbm, p1_hbm, out_hbm,
                  i0_v, i1_v, r0_v, r1_v, s0, s1):
    wid = lax.axis_index("s") * 2 + lax.axis_index("c")
    base = wid * _T_PER_W
    pltpu.sync_copy(p0_hbm.at[0, pl.ds(base, _T_PER_W)], i0_v)
    pltpu.sync_copy(p1_hbm.at[0, pl.ds(base, _T_PER_W)], i1_v)
    cp0 = pltpu.async_copy(ys_hbm.at[i0_v], r0_v, s0)
    cp1 = pltpu.async_copy(ys_hbm.at[i1_v], r1_v, s1)
    cp0.wait()
    cp1.wait()

    def tbody(t, carry):
        for j in range(D // L):
            sl = pl.ds(j * L, L)
            r0_v[t, sl] = r0_v[t, sl] + r1_v[t, sl]
        return carry
    lax.fori_loop(0, _T_PER_W, tbody, 0)

    pltpu.sync_copy(r0_v, out_hbm.at[pl.ds(base, _T_PER_W)])


def kernel(x, Wg, bg, W1, b1, W2, b2):
    f32 = jnp.float32
    i32 = jnp.int32

    pos0, pos1, w0, w1, ob, nb = pl.pallas_call(
        _gate_body,
        out_shape=[
            jax.ShapeDtypeStruct((1, N), i32),
            jax.ShapeDtypeStruct((1, N), i32),
            jax.ShapeDtypeStruct((1, N), f32),
            jax.ShapeDtypeStruct((1, N), f32),
            jax.ShapeDtypeStruct((E, 1), i32),
            jax.ShapeDtypeStruct((E, 1), i32),
        ],
    )(x, Wg, bg.reshape(E, 1))

    grid_spec = pltpu.PrefetchScalarGridSpec(
        num_scalar_prefetch=2,
        grid=(E,),
        in_specs=[
            pl.BlockSpec((1, N), lambda e, ob_s, nb_s: (0, 0)),
            pl.BlockSpec((1, N), lambda e, ob_s, nb_s: (0, 0)),
            pl.BlockSpec((1, N), lambda e, ob_s, nb_s: (0, 0)),
            pl.BlockSpec((1, N), lambda e, ob_s, nb_s: (0, 0)),
            pl.BlockSpec((N, D), lambda e, ob_s, nb_s: (0, 0)),
            pl.BlockSpec((1, D, H), lambda e, ob_s, nb_s: (e, 0, 0)),
            pl.BlockSpec((1, 1, H), lambda e, ob_s, nb_s: (e, 0, 0)),
            pl.BlockSpec((1, H, D), lambda e, ob_s, nb_s: (e, 0, 0)),
            pl.BlockSpec((1, 1, D), lambda e, ob_s, nb_s: (e, 0, 0)),
        ],
        out_specs=pl.BlockSpec(memory_space=pltpu.MemorySpace.HBM),
        scratch_shapes=[
            pltpu.VMEM((2, B, D), f32),
            pltpu.SemaphoreType.DMA((2,)),
        ],
    )
    ys = pl.pallas_call(
        _moe_body,
        grid_spec=grid_spec,
        out_shape=jax.ShapeDtypeStruct((P, D), f32),
    )(ob.reshape(E), nb.reshape(E), pos0, pos1, w0, w1, x,
      W1, b1.reshape(E, 1, H), W2, b2.reshape(E, 1, D))

    mesh = plsc.VectorSubcoreMesh(core_axis_name="c", subcore_axis_name="s")
    combine_k = pl.kernel(
        _combine_body,
        out_type=jax.ShapeDtypeStruct((N, D), f32),
        mesh=mesh,
        scratch_types=[
            pltpu.VMEM((_T_PER_W,), i32),
            pltpu.VMEM((_T_PER_W,), i32),
            pltpu.VMEM((_T_PER_W, D), f32),
            pltpu.VMEM((_T_PER_W, D), f32),
            pltpu.SemaphoreType.DMA,
            pltpu.SemaphoreType.DMA,
        ],
        compiler_params=pltpu.CompilerParams(needs_layout_passes=False),
    )
    return combine_k(ys, pos0, pos1)
